# Initial kernel scaffold; baseline (speedup 1.0000x reference)
import jax
import jax.numpy as jnp
from jax.experimental import pallas as pl


def kernel(x_orig):
    raise NotImplementedError("write your pallas kernel here")


# speedup vs baseline: 28.2627x; 28.2627x over previous
import functools

import jax
import jax.numpy as jnp
from jax import lax
from jax.experimental import pallas as pl
from jax.experimental.pallas import tpu as pltpu
from jax.experimental.pallas import tpu_sc as plsc

_N = 16777216
_BINS = 256
_NC, _NS, _L = 2, 16, 16
_NW = _NC * _NS
_PER_W = _N // _NW
_CHUNK = 32768
_NCHUNK = _PER_W // _CHUNK

_mesh = plsc.VectorSubcoreMesh(
    core_axis_name="c", subcore_axis_name="s",
    num_cores=_NC, num_subcores=_NS)


_MM_GRID = 16
_MM_ROWS = 4096 // _MM_GRID


def _mm_body(x_ref, mn_ref, mx_ref):
    i = pl.program_id(0)
    bmn = jnp.min(x_ref[...])
    bmx = jnp.max(x_ref[...])

    @pl.when(i == 0)
    def _():
        mn_ref[0, 0] = bmn
        mx_ref[0, 0] = bmx

    @pl.when(i != 0)
    def _():
        mn_ref[0, 0] = jnp.minimum(mn_ref[0, 0], bmn)
        mx_ref[0, 0] = jnp.maximum(mx_ref[0, 0], bmx)


_minmax = pl.pallas_call(
    _mm_body,
    grid=(_MM_GRID,),
    in_specs=[pl.BlockSpec((_MM_ROWS, 4096), lambda i: (i, 0))],
    out_specs=[pl.BlockSpec(memory_space=pltpu.SMEM),
               pl.BlockSpec(memory_space=pltpu.SMEM)],
    out_shape=[jax.ShapeDtypeStruct((1, 1), jnp.float32),
               jax.ShapeDtypeStruct((1, 1), jnp.float32)],
)


def _hist_body(x_hbm, par_hbm, out_hbm, buf0, buf1, par_v, hist_v, sem0, sem1):
    wid = lax.axis_index("s") * _NC + lax.axis_index("c")
    base = wid * _PER_W

    pltpu.sync_copy(par_hbm, par_v)
    lo = par_v[pl.ds(0, _L)]
    hi = par_v[pl.ds(_L, _L)]
    span = par_v[pl.ds(2 * _L, _L)]

    lane_base = lax.iota(jnp.int32, _L) * _BINS
    ones = jnp.ones((_L,), jnp.float32)
    zeros = jnp.zeros((_L,), jnp.float32)

    def _zero(j, _):
        hist_v[pl.ds(j * _L, _L)] = zeros
        return 0
    lax.fori_loop(0, (_L * _BINS) // _L, _zero, 0)

    sems = (sem0, sem1)
    bufs = (buf0, buf1)
    hdl = [None, None]
    hdl[0] = pltpu.async_copy(x_hbm.at[pl.ds(base, _CHUNK)], buf0, sem0)
    for c in range(_NCHUNK):
        b = c & 1
        hdl[b].wait()
        if c + 1 < _NCHUNK:
            nb = 1 - b
            hdl[nb] = pltpu.async_copy(
                x_hbm.at[pl.ds(base + (c + 1) * _CHUNK, _CHUNK)],
                bufs[nb], sems[nb])
        bb = bufs[b]

        def _vstep(i, _):
            v = bb[pl.ds(i * _L, _L)]
            t = ((v - lo) * jnp.float32(_BINS)) / span
            ii = t.astype(jnp.int32)
            inr = (v >= lo) & (v <= hi)
            idx = jnp.where(v == hi, _BINS - 1, ii)
            idx = jnp.clip(idx, 0, _BINS - 1)
            plsc.addupdate_scatter(hist_v, [lane_base + idx], ones, mask=inr)
            return 0
        lax.fori_loop(0, _CHUNK // _L, _vstep, 0)

    def _rrow(r, _):
        def _rcol(j, _2):
            acc = hist_v[pl.ds(j * _L, _L)]
            add = hist_v[pl.ds(r * _BINS + j * _L, _L)]
            hist_v[pl.ds(j * _L, _L)] = acc + add
            return 0
        lax.fori_loop(0, _BINS // _L, _rcol, 0)
        return 0
    lax.fori_loop(1, _L, _rrow, 0)

    pltpu.sync_copy(hist_v.at[pl.ds(0, _BINS)], out_hbm.at[wid])


_hist = functools.partial(
    pl.kernel,
    out_type=jax.ShapeDtypeStruct((_NW, _BINS), jnp.float32),
    mesh=_mesh,
    compiler_params=pltpu.CompilerParams(needs_layout_passes=False),
    scratch_types=[
        pltpu.VMEM((_CHUNK,), jnp.float32),
        pltpu.VMEM((_CHUNK,), jnp.float32),
        pltpu.VMEM((3 * _L,), jnp.float32),
        pltpu.VMEM((_L * _BINS,), jnp.float32),
        pltpu.SemaphoreType.DMA,
        pltpu.SemaphoreType.DMA,
    ],
)(_hist_body)


def kernel(x_orig):
    x = lax.stop_gradient(x_orig)
    mn, mx = _minmax(x.reshape(4096, 4096))
    mn_s = mn[0, 0]
    mx_s = mx[0, 0]
    lo = jnp.trunc(mn_s)
    hi = jnp.trunc(mx_s)
    span = hi - lo
    safe = jnp.where(span == 0, jnp.float32(1.0), span)
    params = jnp.concatenate([
        jnp.full((_L,), lo, jnp.float32),
        jnp.full((_L,), hi, jnp.float32),
        jnp.full((_L,), safe, jnp.float32),
    ])
    parts = _hist(x, params)
    histogram = jnp.sum(parts, axis=0)
    return x_orig, histogram, mn_s, mx_s


# speedup vs baseline: 79.2616x; 2.8045x over previous
import functools

import jax
import jax.numpy as jnp
from jax import lax
from jax.experimental import pallas as pl
from jax.experimental.pallas import tpu as pltpu
from jax.experimental.pallas import tpu_sc as plsc

_N = 16777216
_BINS = 256
_NC, _NS, _L = 2, 16, 16
_NW = _NC * _NS
_PER_W = _N // _NW
_CHUNK = 32768
_NCHUNK = _PER_W // _CHUNK

_mesh = plsc.VectorSubcoreMesh(
    core_axis_name="c", subcore_axis_name="s",
    num_cores=_NC, num_subcores=_NS)


_MM_GRID = 16
_MM_ROWS = 4096 // _MM_GRID


def _mm_body(x_ref, mn_ref, mx_ref):
    i = pl.program_id(0)
    bmn = jnp.min(x_ref[...])
    bmx = jnp.max(x_ref[...])

    @pl.when(i == 0)
    def _():
        mn_ref[0, 0] = bmn
        mx_ref[0, 0] = bmx

    @pl.when(i != 0)
    def _():
        mn_ref[0, 0] = jnp.minimum(mn_ref[0, 0], bmn)
        mx_ref[0, 0] = jnp.maximum(mx_ref[0, 0], bmx)


_minmax = pl.pallas_call(
    _mm_body,
    grid=(_MM_GRID,),
    in_specs=[pl.BlockSpec((_MM_ROWS, 4096), lambda i: (i, 0))],
    out_specs=[pl.BlockSpec(memory_space=pltpu.SMEM),
               pl.BlockSpec(memory_space=pltpu.SMEM)],
    out_shape=[jax.ShapeDtypeStruct((1, 1), jnp.float32),
               jax.ShapeDtypeStruct((1, 1), jnp.float32)],
)


def _hist_body(x_hbm, par_hbm, out_hbm, buf0, buf1, par_v, hist_v, sem0, sem1):
    wid = lax.axis_index("s") * _NC + lax.axis_index("c")
    base = wid * _PER_W

    pltpu.sync_copy(par_hbm, par_v)
    lo = par_v[pl.ds(0, _L)]
    hi = par_v[pl.ds(_L, _L)]
    scale = par_v[pl.ds(2 * _L, _L)]

    lane_base = lax.iota(jnp.int32, _L) * _BINS
    ones = jnp.ones((_L,), jnp.float32)
    zeros = jnp.zeros((_L,), jnp.float32)

    def _zero(j, _):
        hist_v[pl.ds(j * _L, _L)] = zeros
        return 0
    lax.fori_loop(0, (_L * _BINS) // _L, _zero, 0)

    sems = (sem0, sem1)
    bufs = (buf0, buf1)
    hdl = [None, None]
    hdl[0] = pltpu.async_copy(x_hbm.at[pl.ds(base, _CHUNK)], buf0, sem0)
    for c in range(_NCHUNK):
        b = c & 1
        hdl[b].wait()
        if c + 1 < _NCHUNK:
            nb = 1 - b
            hdl[nb] = pltpu.async_copy(
                x_hbm.at[pl.ds(base + (c + 1) * _CHUNK, _CHUNK)],
                bufs[nb], sems[nb])
        bb = bufs[b]

        @plsc.parallel_loop(0, _CHUNK, _L, unroll=8)
        def _vstep(i):
            v = bb[pl.ds(i, _L)]
            t = (v - lo) * scale
            ii = t.astype(jnp.int32)
            idx = jnp.minimum(ii, _BINS - 1)
            inr = (v >= lo) & (v <= hi)
            plsc.addupdate_scatter(hist_v, [lane_base + idx], ones, mask=inr)

    def _rrow(r, _):
        def _rcol(j, _2):
            acc = hist_v[pl.ds(j * _L, _L)]
            add = hist_v[pl.ds(r * _BINS + j * _L, _L)]
            hist_v[pl.ds(j * _L, _L)] = acc + add
            return 0
        lax.fori_loop(0, _BINS // _L, _rcol, 0)
        return 0
    lax.fori_loop(1, _L, _rrow, 0)

    pltpu.sync_copy(hist_v.at[pl.ds(0, _BINS)], out_hbm.at[wid])


_hist = functools.partial(
    pl.kernel,
    out_type=jax.ShapeDtypeStruct((_NW, _BINS), jnp.float32),
    mesh=_mesh,
    compiler_params=pltpu.CompilerParams(needs_layout_passes=False),
    scratch_types=[
        pltpu.VMEM((_CHUNK,), jnp.float32),
        pltpu.VMEM((_CHUNK,), jnp.float32),
        pltpu.VMEM((3 * _L,), jnp.float32),
        pltpu.VMEM((_L * _BINS,), jnp.float32),
        pltpu.SemaphoreType.DMA,
        pltpu.SemaphoreType.DMA,
    ],
)(_hist_body)


def kernel(x_orig):
    x = lax.stop_gradient(x_orig)
    mn, mx = _minmax(x.reshape(4096, 4096))
    mn_s = mn[0, 0]
    mx_s = mx[0, 0]
    lo = jnp.trunc(mn_s)
    hi = jnp.trunc(mx_s)
    span = hi - lo
    safe = jnp.where(span == 0, jnp.float32(1.0), span)
    params = jnp.concatenate([
        jnp.full((_L,), lo, jnp.float32),
        jnp.full((_L,), hi, jnp.float32),
        jnp.full((_L,), jnp.float32(_BINS) / safe, jnp.float32),
    ])
    parts = _hist(x, params)
    histogram = jnp.sum(parts, axis=0)
    return x_orig, histogram, mn_s, mx_s


# speedup vs baseline: 101.5444x; 1.2811x over previous
import functools

import jax
import jax.numpy as jnp
from jax import lax
from jax.experimental import pallas as pl
from jax.experimental.pallas import tpu as pltpu
from jax.experimental.pallas import tpu_sc as plsc

_N = 16777216
_BINS = 256
_NC, _NS, _L = 2, 16, 16
_NW = _NC * _NS
_PER_W = _N // _NW
_CHUNK = 32768
_NCHUNK = _PER_W // _CHUNK

_mesh = plsc.VectorSubcoreMesh(
    core_axis_name="c", subcore_axis_name="s",
    num_cores=_NC, num_subcores=_NS)


_MM_GRID = 16
_MM_BLK = _N // _MM_GRID


def _mm_body(x_ref, mn_ref, mx_ref, xout_ref):
    i = pl.program_id(0)
    v = x_ref[...]
    xout_ref[...] = v
    bmn = jnp.min(v)
    bmx = jnp.max(v)

    @pl.when(i == 0)
    def _():
        mn_ref[0, 0] = bmn
        mx_ref[0, 0] = bmx

    @pl.when(i != 0)
    def _():
        mn_ref[0, 0] = jnp.minimum(mn_ref[0, 0], bmn)
        mx_ref[0, 0] = jnp.maximum(mx_ref[0, 0], bmx)


_minmax = pl.pallas_call(
    _mm_body,
    grid=(_MM_GRID,),
    in_specs=[pl.BlockSpec((_MM_BLK,), lambda i: (i,))],
    out_specs=[pl.BlockSpec(memory_space=pltpu.SMEM),
               pl.BlockSpec(memory_space=pltpu.SMEM),
               pl.BlockSpec((_MM_BLK,), lambda i: (i,))],
    out_shape=[jax.ShapeDtypeStruct((1, 1), jnp.float32),
               jax.ShapeDtypeStruct((1, 1), jnp.float32),
               jax.ShapeDtypeStruct((_N,), jnp.float32)],
)


def _hist_body(x_hbm, par_hbm, out_hbm, buf0, buf1, par_v, hist_v, sem0, sem1):
    wid = lax.axis_index("s") * _NC + lax.axis_index("c")
    base = wid * _PER_W

    pltpu.sync_copy(par_hbm, par_v)
    lo = par_v[pl.ds(0, _L)]
    hi = par_v[pl.ds(_L, _L)]
    scale = par_v[pl.ds(2 * _L, _L)]

    lane_base = lax.iota(jnp.int32, _L) * _BINS
    ones = jnp.ones((_L,), jnp.float32)
    zeros = jnp.zeros((_L,), jnp.float32)

    def _zero(j, _):
        hist_v[pl.ds(j * _L, _L)] = zeros
        return 0
    lax.fori_loop(0, (_L * _BINS) // _L, _zero, 0)

    sems = (sem0, sem1)
    bufs = (buf0, buf1)
    hdl = [None, None]
    hdl[0] = pltpu.async_copy(x_hbm.at[pl.ds(base, _CHUNK)], buf0, sem0)
    for c in range(_NCHUNK):
        b = c & 1
        hdl[b].wait()
        if c + 1 < _NCHUNK:
            nb = 1 - b
            hdl[nb] = pltpu.async_copy(
                x_hbm.at[pl.ds(base + (c + 1) * _CHUNK, _CHUNK)],
                bufs[nb], sems[nb])
        bb = bufs[b]

        @plsc.parallel_loop(0, _CHUNK, _L, unroll=8)
        def _vstep(i):
            v = bb[pl.ds(i, _L)]
            t = (v - lo) * scale
            ii = t.astype(jnp.int32)
            idx = jnp.minimum(ii, _BINS - 1)
            inr = (v >= lo) & (v <= hi)
            plsc.addupdate_scatter(hist_v, [lane_base + idx], ones, mask=inr)

    def _rrow(r, _):
        def _rcol(j, _2):
            acc = hist_v[pl.ds(j * _L, _L)]
            add = hist_v[pl.ds(r * _BINS + j * _L, _L)]
            hist_v[pl.ds(j * _L, _L)] = acc + add
            return 0
        lax.fori_loop(0, _BINS // _L, _rcol, 0)
        return 0
    lax.fori_loop(1, _L, _rrow, 0)

    pltpu.sync_copy(hist_v.at[pl.ds(0, _BINS)], out_hbm.at[wid])


_hist = functools.partial(
    pl.kernel,
    out_type=jax.ShapeDtypeStruct((_NW, _BINS), jnp.float32),
    mesh=_mesh,
    compiler_params=pltpu.CompilerParams(needs_layout_passes=False),
    scratch_types=[
        pltpu.VMEM((_CHUNK,), jnp.float32),
        pltpu.VMEM((_CHUNK,), jnp.float32),
        pltpu.VMEM((3 * _L,), jnp.float32),
        pltpu.VMEM((_L * _BINS,), jnp.float32),
        pltpu.SemaphoreType.DMA,
        pltpu.SemaphoreType.DMA,
    ],
)(_hist_body)


def kernel(x_orig):
    x = lax.stop_gradient(x_orig)
    mn, mx, x_out = _minmax(x)
    mn_s = mn[0, 0]
    mx_s = mx[0, 0]
    lo = jnp.trunc(mn_s)
    hi = jnp.trunc(mx_s)
    span = hi - lo
    safe = jnp.where(span == 0, jnp.float32(1.0), span)
    params = jnp.concatenate([
        jnp.full((_L,), lo, jnp.float32),
        jnp.full((_L,), hi, jnp.float32),
        jnp.full((_L,), jnp.float32(_BINS) / safe, jnp.float32),
    ])
    parts = _hist(x, params)
    histogram = jnp.sum(parts, axis=0)
    return x_out, histogram, mn_s, mx_s
